# Initial kernel scaffold; baseline (speedup 1.0000x reference)
import jax
import jax.numpy as jnp
from jax.experimental import pallas as pl


def kernel(feat_, coor_, batch, We, be, W1, b1, W2, b2, Wg, bg, Wc1, bc1, Wc2, bc2, Wn1, bn1, Wn2, bn2, scale):
    raise NotImplementedError("write your pallas kernel here")


# speedup vs baseline: 3.7597x; 3.7597x over previous
import functools

import jax
import jax.numpy as jnp
from jax import lax
from jax.experimental import pallas as pl
from jax.experimental.pallas import tpu as pltpu
from jax.experimental.pallas import tpu_sc as plsc

LAYERS = 3
DIM = 128
POS = 3
N = 2048
K = 128
EIN = 2 * DIM + 1
H1 = 2 * EIN
DG = DIM + 16
NW = 32
ROWS_PER_W = N // NW
R1 = 256
R3 = 8
F32 = jnp.float32


def _lk(x):
    return jnp.maximum(x, 0.1 * x)


def _embed_body(f_ref, we_ref, be_ref, o_ref):
    o_ref[...] = _lk(
        jnp.dot(f_ref[...], we_ref[...], preferred_element_type=F32)
        + be_ref[...])


def _embed(feat, We, be2d):
    return pl.pallas_call(
        _embed_body,
        grid=(N // R1,),
        in_specs=[
            pl.BlockSpec((R1, DIM), lambda i: (i, 0)),
            pl.BlockSpec((DIM, DIM), lambda i: (0, 0)),
            pl.BlockSpec((1, DIM), lambda i: (0, 0)),
        ],
        out_specs=pl.BlockSpec((R1, DIM), lambda i: (i, 0)),
        out_shape=jax.ShapeDtypeStruct((N, DIM), F32),
    )(feat, We, be2d)


def _s1_body(c_ref, ct_ref, f_ref, w1a_ref, b1_ref, dist_ref, thr_ref, a_ref):
    c = c_ref[...]
    ct = ct_ref[...]
    d = ((c[:, 0:1] - ct[0:1, :]) ** 2
         + (c[:, 1:2] - ct[1:2, :]) ** 2
         + (c[:, 2:3] - ct[2:3, :]) ** 2)
    dist_ref[...] = d

    bits = lax.bitcast_convert_type(d, jnp.int32)
    lo0 = jnp.zeros((R1, 1), jnp.int32)
    hi0 = jnp.full((R1, 1), 0x7F800000, jnp.int32)

    def step(_, carry):
        lo, hi = carry
        mid = lo + ((hi - lo) >> 1)
        cnt = jnp.sum((bits <= mid).astype(jnp.int32), axis=1, keepdims=True)
        ok = cnt >= K
        return jnp.where(ok, lo, mid + 1), jnp.where(ok, mid, hi)

    _, hi = lax.fori_loop(0, 31, step, (lo0, hi0))
    thr = lax.bitcast_convert_type(hi, F32)
    thr_ref[...] = jnp.broadcast_to(thr, (R1, 16))

    a_ref[...] = (jnp.dot(f_ref[...], w1a_ref[...],
                          preferred_element_type=F32) + b1_ref[...])


def _s1(coors, coorsT, feats, W1a, b1_2d):
    return pl.pallas_call(
        _s1_body,
        grid=(N // R1,),
        in_specs=[
            pl.BlockSpec((R1, 4), lambda i: (i, 0)),
            pl.BlockSpec((4, N), lambda i: (0, 0)),
            pl.BlockSpec((R1, DIM), lambda i: (i, 0)),
            pl.BlockSpec((DIM, H1), lambda i: (0, 0)),
            pl.BlockSpec((1, H1), lambda i: (0, 0)),
        ],
        out_specs=[
            pl.BlockSpec((R1, N), lambda i: (i, 0)),
            pl.BlockSpec((R1, 16), lambda i: (i, 0)),
            pl.BlockSpec((R1, H1), lambda i: (i, 0)),
        ],
        out_shape=[
            jax.ShapeDtypeStruct((N, N), F32),
            jax.ShapeDtypeStruct((N, 16), F32),
            jax.ShapeDtypeStruct((N, H1), F32),
        ],
    )(coors, coorsT, feats, W1a, b1_2d)


def _sc_body(dist_hbm, thr_hbm, feats_hbm, coorsT_hbm, gf_hbm, gc_hbm,
             dist_v, thr_v, ltbuf, eqbuf, idxbuf, rows_v, cx_v, cy_v, cz_v,
             cbuf, sem):
    wid = lax.axis_index("s") * 2 + lax.axis_index("c")
    row0 = wid * ROWS_PER_W

    pltpu.sync_copy(coorsT_hbm.at[0], cx_v)
    pltpu.sync_copy(coorsT_hbm.at[1], cy_v)
    pltpu.sync_copy(coorsT_hbm.at[2], cz_v)

    def do_row(r, _):
        row = row0 + r
        pltpu.sync_copy(dist_hbm.at[row], dist_v)
        pltpu.sync_copy(thr_hbm.at[row], thr_v)
        t = thr_v[...]

        zero16 = jnp.zeros((16,), jnp.int32)

        @plsc.parallel_loop(0, N // 16, 1, unroll=4, carry=(zero16, zero16))
        def chunk(cc, carry):
            nlt_v, neq_v = carry
            d = dist_v[pl.ds(cc * 16, 16)]
            col = lax.iota(jnp.int32, 16) + cc * 16
            is_lt = d < t
            is_eq = d == t
            clt = plsc.cumsum(jnp.where(is_lt, 1, 0))
            ceq = plsc.cumsum(jnp.where(is_eq, 1, 0))
            plsc.store_scatter(ltbuf, [nlt_v + clt - 1], col, mask=is_lt)
            plsc.store_scatter(eqbuf, [neq_v + ceq - 1], col, mask=is_eq)
            nlt_v = nlt_v + plsc.all_reduce_population_count(is_lt)
            neq_v = neq_v + plsc.all_reduce_population_count(is_eq)
            return nlt_v, neq_v

        nlt_v, _ = chunk

        iot = lax.iota(jnp.int32, 16)
        for kc in range(K // 16):
            v = eqbuf[pl.ds(kc * 16, 16)]
            plsc.store_scatter(ltbuf, [nlt_v + kc * 16 + iot], v)

        for kc in range(K // 16):
            idx = ltbuf[pl.ds(kc * 16, 16)]
            idxbuf[pl.ds(kc * 16, 16)] = idx
            cbuf[0, pl.ds(kc * 16, 16)] = plsc.load_gather(cx_v, [idx])
            cbuf[1, pl.ds(kc * 16, 16)] = plsc.load_gather(cy_v, [idx])
            cbuf[2, pl.ds(kc * 16, 16)] = plsc.load_gather(cz_v, [idx])

        pltpu.async_copy(feats_hbm.at[idxbuf], rows_v, sem).wait()
        pltpu.sync_copy(rows_v, gf_hbm.at[pl.ds(row * K, K)])
        pltpu.sync_copy(cbuf, gc_hbm.at[row])
        return 0

    lax.fori_loop(0, ROWS_PER_W, do_row, 0)


def _s2(dist, thr, feats, coorsT):
    mesh = plsc.VectorSubcoreMesh(core_axis_name="c", subcore_axis_name="s")
    fn = functools.partial(
        pl.kernel, mesh=mesh,
        compiler_params=pltpu.CompilerParams(needs_layout_passes=False),
        out_type=[
            jax.ShapeDtypeStruct((N * K, DIM), F32),
            jax.ShapeDtypeStruct((N, 4, K), F32),
        ],
        scratch_types=[
            pltpu.VMEM((N,), F32),
            pltpu.VMEM((16,), F32),
            pltpu.VMEM((272,), jnp.int32),
            pltpu.VMEM((N + 16,), jnp.int32),
            pltpu.VMEM((K,), jnp.int32),
            pltpu.VMEM((K, DIM), F32),
            pltpu.VMEM((N,), F32),
            pltpu.VMEM((N,), F32),
            pltpu.VMEM((N,), F32),
            pltpu.VMEM((4, K), F32),
            pltpu.SemaphoreType.DMA,
        ],
    )(_sc_body)
    return fn(dist, thr, feats, coorsT)


def _s3_body(with_coors, g_ref, gc_ref, a_ref, f_ref, c_ref,
             w1b_ref, w1d_ref, w2_ref, b2_ref, wgT_ref, bg_ref,
             wc1_ref, bc1_ref, wc2T_ref, bc2_ref, sc_ref,
             wn1_ref, bn1_ref, wn2_ref, bn2_ref,
             fout_ref, cout_ref, mi_ref):
    ii = lax.broadcasted_iota(jnp.int32, (K, K), 0)
    jj = lax.broadcasted_iota(jnp.int32, (K, K), 1)
    eye = ii == jj
    eye4 = (lax.broadcasted_iota(jnp.int32, (4, 4), 0)
            == lax.broadcasted_iota(jnp.int32, (4, 4), 1))

    rels = []
    rd_rows = []
    rd_cols = []
    for n in range(R3):
        cj = gc_ref[n, :, :]
        ci_row = c_ref[pl.ds(n, 1), :]
        ci_col = jnp.sum(
            jnp.where(eye4, jnp.broadcast_to(ci_row, (4, 4)), 0.0),
            axis=1, keepdims=True)
        rel = ci_col - cj
        rx, ry, rz = rel[0:1, :], rel[1:2, :], rel[2:3, :]
        rd_row = rx * rx + ry * ry + rz * rz
        rd_col = jnp.sum(
            jnp.where(eye, jnp.broadcast_to(rd_row, (K, K)), 0.0),
            axis=1, keepdims=True)
        rels.append((rx, ry, rz))
        rd_rows.append(rd_row)
        rd_cols.append(rd_col)

    a_stack = jnp.concatenate(
        [jnp.broadcast_to(a_ref[pl.ds(n, 1), :], (K, H1)) for n in range(R3)],
        axis=0)
    rd_stack = jnp.concatenate(rd_cols, axis=0)
    h = _lk(jnp.dot(g_ref[...], w1b_ref[...], preferred_element_type=F32)
            + a_stack + rd_stack * w1d_ref[...])
    m = _lk(jnp.dot(h, w2_ref[...], preferred_element_type=F32)
            + b2_ref[...])
    glog = (jnp.sum(m * wgT_ref[...], axis=1, keepdims=True)
            + bg_ref[0])
    m = m * jax.nn.sigmoid(glog)
    if with_coors:
        ch = _lk(jnp.dot(m, wc1_ref[...], preferred_element_type=F32)
                 + bc1_ref[...])
        cw = (jnp.sum(ch * wc2T_ref[...], axis=1, keepdims=True)
              + bc2_ref[0])

    for n in range(R3):
        mi_ref[pl.ds(n, 1), :] = jnp.sum(m[n * K:(n + 1) * K, :], axis=0,
                                         keepdims=True)
        if with_coors:
            rx, ry, rz = rels[n]
            cw_col = cw[n * K:(n + 1) * K, :]
            cw_row = jnp.sum(
                jnp.where(eye, jnp.broadcast_to(cw_col, (K, K)), 0.0),
                axis=0, keepdims=True)
            inv = sc_ref[0] / jnp.clip(
                jnp.sqrt(jnp.clip(rd_rows[n], 1e-12, None)), 1e-8, None)
            w = cw_row * inv
            dx = jnp.sum(w * rx, axis=1, keepdims=True)
            dy = jnp.sum(w * ry, axis=1, keepdims=True)
            dz = jnp.sum(w * rz, axis=1, keepdims=True)
            delta = jnp.concatenate(
                [dx, dy, dz, jnp.zeros((1, 1), F32)], axis=1)
            cout_ref[pl.ds(n, 1), :] = c_ref[pl.ds(n, 1), :] + delta
        else:
            cout_ref[pl.ds(n, 1), :] = c_ref[pl.ds(n, 1), :]

    x = jnp.concatenate([f_ref[...], mi_ref[...]], axis=1)
    nh = _lk(jnp.dot(x, wn1_ref[...], preferred_element_type=F32)
             + bn1_ref[...])
    fout_ref[...] = (jnp.dot(nh, wn2_ref[...], preferred_element_type=F32)
                     + bn2_ref[...] + f_ref[...])


def _s3(with_coors, G, GC, A, feats, coors, w1b, w1d, w2, b2,
        wgT, bg, wc1, bc1, wc2T, bc2, sc, wn1, bn1, wn2, bn2):
    full = lambda shape: pl.BlockSpec(shape, lambda i: tuple(0 for _ in shape))
    smem = pl.BlockSpec(memory_space=pltpu.SMEM)
    return pl.pallas_call(
        functools.partial(_s3_body, with_coors),
        grid=(N // R3,),
        in_specs=[
            pl.BlockSpec((R3 * K, DIM), lambda i: (i, 0)),
            pl.BlockSpec((R3, 4, K), lambda i: (i, 0, 0)),
            pl.BlockSpec((R3, H1), lambda i: (i, 0)),
            pl.BlockSpec((R3, DIM), lambda i: (i, 0)),
            pl.BlockSpec((R3, 4), lambda i: (i, 0)),
            full((DIM, H1)), full((1, H1)),
            full((H1, DIM)), full((1, DIM)), full((1, DIM)), smem,
            full((DIM, 4 * DIM)), full((1, 4 * DIM)), full((1, 4 * DIM)),
            smem, smem,
            full((2 * DIM, 2 * DIM)), full((1, 2 * DIM)),
            full((2 * DIM, DIM)), full((1, DIM)),
        ],
        out_specs=[
            pl.BlockSpec((R3, DIM), lambda i: (i, 0)),
            pl.BlockSpec((R3, 4), lambda i: (i, 0)),
        ],
        out_shape=[
            jax.ShapeDtypeStruct((N, DIM), F32),
            jax.ShapeDtypeStruct((N, 4), F32),
        ],
        scratch_shapes=[pltpu.VMEM((R3, DIM), F32)],
    )(G, GC, A, feats, coors, w1b, w1d, w2, b2, wgT, bg,
      wc1, bc1, wc2T, bc2, sc, wn1, bn1, wn2, bn2)


def kernel(feat_, coor_, batch, We, be, W1, b1, W2, b2, Wg, bg, Wc1, bc1,
           Wc2, bc2, Wn1, bn1, Wn2, bn2, scale):
    feats = _embed(feat_, We, be.reshape(1, DIM))
    coors = jnp.concatenate([coor_, jnp.zeros((N, 1), F32)], axis=1)

    for i in range(LAYERS):
        W1a = W1[i, :DIM]
        W1b = W1[i, DIM:2 * DIM]
        w1d = W1[i, 2 * DIM:2 * DIM + 1]
        coorsT = coors.T
        dist, thr, A = _s1(coors, coorsT, feats, W1a, b1[i].reshape(1, H1))
        G, GC = _s2(dist, thr, feats, coorsT)
        feats, coors = _s3(
            i < LAYERS - 1, G, GC, A, feats, coors,
            W1b, w1d,
            W2[i], b2[i].reshape(1, DIM), Wg[i].T, bg[i],
            Wc1[i], bc1[i].reshape(1, 4 * DIM), Wc2[i].T, bc2[i], scale[i],
            Wn1[i], bn1[i].reshape(1, 2 * DIM),
            Wn2[i], bn2[i].reshape(1, DIM))
    return feats


# speedup vs baseline: 4.3898x; 1.1676x over previous
import functools

import jax
import jax.numpy as jnp
from jax import lax
from jax.experimental import pallas as pl
from jax.experimental.pallas import tpu as pltpu
from jax.experimental.pallas import tpu_sc as plsc

LAYERS = 3
DIM = 128
POS = 3
N = 2048
K = 128
EIN = 2 * DIM + 1
H1 = 2 * EIN
DG = DIM + 16
NW = 32
ROWS_PER_W = N // NW
R1 = 256
R3 = 8
F32 = jnp.float32


def _lk(x):
    return jnp.maximum(x, 0.1 * x)


def _embed_body(f_ref, we_ref, be_ref, o_ref):
    o_ref[...] = _lk(
        jnp.dot(f_ref[...], we_ref[...], preferred_element_type=F32)
        + be_ref[...])


def _embed(feat, We, be2d):
    return pl.pallas_call(
        _embed_body,
        grid=(N // R1,),
        in_specs=[
            pl.BlockSpec((R1, DIM), lambda i: (i, 0)),
            pl.BlockSpec((DIM, DIM), lambda i: (0, 0)),
            pl.BlockSpec((1, DIM), lambda i: (0, 0)),
        ],
        out_specs=pl.BlockSpec((R1, DIM), lambda i: (i, 0)),
        out_shape=jax.ShapeDtypeStruct((N, DIM), F32),
    )(feat, We, be2d)


def _s1_body(c_ref, ct_ref, f_ref, w1a_ref, b1_ref, dist_ref, a_ref):
    c = c_ref[...]
    ct = ct_ref[...]
    d = ((c[:, 0:1] - ct[0:1, :]) ** 2
         + (c[:, 1:2] - ct[1:2, :]) ** 2
         + (c[:, 2:3] - ct[2:3, :]) ** 2)

    bits = lax.bitcast_convert_type(d, jnp.int32)
    lo0 = jnp.zeros((R1, 1), jnp.int32)
    hi0 = jnp.full((R1, 1), 0x7F800000, jnp.int32)
    def step(_, carry):
        lo, hi = carry
        mid = lo + ((hi - lo) >> 1)
        cnt = jnp.sum((bits <= mid).astype(jnp.int32), axis=1, keepdims=True)
        ok = cnt >= K
        return jnp.where(ok, lo, mid + 1), jnp.where(ok, mid, hi)

    _, hi = lax.fori_loop(0, 31, step, (lo0, hi0))
    dist_ref[...] = bits - hi

    a_ref[...] = (jnp.dot(f_ref[...], w1a_ref[...],
                          preferred_element_type=F32) + b1_ref[...])


def _s1(coors, coorsT, feats, W1a, b1_2d):
    return pl.pallas_call(
        _s1_body,
        grid=(N // R1,),
        in_specs=[
            pl.BlockSpec((R1, 4), lambda i: (i, 0)),
            pl.BlockSpec((4, N), lambda i: (0, 0)),
            pl.BlockSpec((R1, DIM), lambda i: (i, 0)),
            pl.BlockSpec((DIM, H1), lambda i: (0, 0)),
            pl.BlockSpec((1, H1), lambda i: (0, 0)),
        ],
        out_specs=[
            pl.BlockSpec((R1, N), lambda i: (i, 0)),
            pl.BlockSpec((R1, H1), lambda i: (i, 0)),
        ],
        out_shape=[
            jax.ShapeDtypeStruct((N, N), jnp.int32),
            jax.ShapeDtypeStruct((N, H1), F32),
        ],
    )(coors, coorsT, feats, W1a, b1_2d)


def _sc_body(dist_hbm, feats_hbm, coorsT_hbm, gf_hbm, gc_hbm,
             dist_v, ltbuf, eqbuf, idxbuf, rows_v, cx_v, cy_v, cz_v,
             cbuf, ld0, ld1, g0, g1, st0, st1, sc0, sc1):
    wid = lax.axis_index("s") * 2 + lax.axis_index("c")
    row0 = wid * ROWS_PER_W
    ld = (ld0, ld1)
    gs = (g0, g1)
    st = (st0, st1)
    sc = (sc0, sc1)
    iot = lax.iota(jnp.int32, 16)
    zero16 = jnp.zeros((16,), jnp.int32)

    pltpu.sync_copy(coorsT_hbm.at[0], cx_v)
    pltpu.sync_copy(coorsT_hbm.at[1], cy_v)
    pltpu.sync_copy(coorsT_hbm.at[2], cz_v)

    pltpu.async_copy(dist_hbm.at[row0], dist_v.at[0], ld[0])
    pltpu.async_copy(dist_hbm.at[row0 + 1], dist_v.at[1], ld[1])

    def finish_row(k, b):
        row = row0 + k
        pltpu.make_async_copy(feats_hbm.at[idxbuf.at[b]], rows_v.at[b],
                              gs[b]).wait()

        @pl.when(k >= 2)
        def _():
            pltpu.make_async_copy(cbuf.at[b], gc_hbm.at[row], sc[b]).wait()

        for kc in range(K // 16):
            idx = idxbuf[b, pl.ds(kc * 16, 16)]
            cbuf[b, 0, pl.ds(kc * 16, 16)] = plsc.load_gather(cx_v, [idx])
            cbuf[b, 1, pl.ds(kc * 16, 16)] = plsc.load_gather(cy_v, [idx])
            cbuf[b, 2, pl.ds(kc * 16, 16)] = plsc.load_gather(cz_v, [idx])

        pltpu.async_copy(rows_v.at[b], gf_hbm.at[pl.ds(row * K, K)], st[b])
        pltpu.async_copy(cbuf.at[b], gc_hbm.at[row], sc[b])

    def row_step(k, b):
        row = row0 + k
        pltpu.make_async_copy(dist_hbm.at[row], dist_v.at[b], ld[b]).wait()

        @plsc.parallel_loop(0, N // 16, 1, unroll=4, carry=(zero16, zero16))
        def chunk(cc, carry):
            nlt_v, neq_v = carry
            d = dist_v[b, pl.ds(cc * 16, 16)]
            col = iot + cc * 16
            is_lt = d < 0
            is_eq = d == 0
            clt = plsc.cumsum(jnp.where(is_lt, 1, 0))
            ceq = plsc.cumsum(jnp.where(is_eq, 1, 0))
            plsc.store_scatter(ltbuf, [nlt_v + clt - 1], col, mask=is_lt)
            plsc.store_scatter(eqbuf, [neq_v + ceq - 1], col, mask=is_eq)
            nlt_v = nlt_v + plsc.all_reduce_population_count(is_lt)
            neq_v = neq_v + plsc.all_reduce_population_count(is_eq)
            return nlt_v, neq_v

        nlt_v, _ = chunk

        for kc in range(K // 16):
            v = eqbuf[pl.ds(kc * 16, 16)]
            plsc.store_scatter(ltbuf, [nlt_v + kc * 16 + iot], v)
        for kc in range(K // 16):
            idxbuf[b, pl.ds(kc * 16, 16)] = ltbuf[pl.ds(kc * 16, 16)]

        @pl.when(k >= 2)
        def _():
            pltpu.make_async_copy(rows_v.at[b], gf_hbm.at[pl.ds(0, K)],
                                  st[b]).wait()

        pltpu.async_copy(feats_hbm.at[idxbuf.at[b]], rows_v.at[b], gs[b])

        @pl.when(k + 2 < ROWS_PER_W)
        def _():
            pltpu.async_copy(dist_hbm.at[row + 2], dist_v.at[b], ld[b])

        @pl.when(k >= 1)
        def _():
            finish_row(k - 1, b ^ 1)

    def pair(i, _):
        row_step(2 * i, 0)
        row_step(2 * i + 1, 1)
        return 0

    lax.fori_loop(0, ROWS_PER_W // 2, pair, 0)
    finish_row(ROWS_PER_W - 1, 1)
    pltpu.make_async_copy(rows_v.at[0], gf_hbm.at[pl.ds(0, K)], st[0]).wait()
    pltpu.make_async_copy(rows_v.at[1], gf_hbm.at[pl.ds(0, K)], st[1]).wait()
    pltpu.make_async_copy(cbuf.at[0], gc_hbm.at[0], sc[0]).wait()
    pltpu.make_async_copy(cbuf.at[1], gc_hbm.at[0], sc[1]).wait()


def _s2(dist, feats, coorsT):
    mesh = plsc.VectorSubcoreMesh(core_axis_name="c", subcore_axis_name="s")
    fn = functools.partial(
        pl.kernel, mesh=mesh,
        compiler_params=pltpu.CompilerParams(needs_layout_passes=False),
        out_type=[
            jax.ShapeDtypeStruct((N * K, DIM), F32),
            jax.ShapeDtypeStruct((N, 4, K), F32),
        ],
        scratch_types=[
            pltpu.VMEM((2, N), jnp.int32),
            pltpu.VMEM((272,), jnp.int32),
            pltpu.VMEM((N + 16,), jnp.int32),
            pltpu.VMEM((2, K), jnp.int32),
            pltpu.VMEM((2, K, DIM), F32),
            pltpu.VMEM((N,), F32),
            pltpu.VMEM((N,), F32),
            pltpu.VMEM((N,), F32),
            pltpu.VMEM((2, 4, K), F32),
            pltpu.SemaphoreType.DMA,
            pltpu.SemaphoreType.DMA,
            pltpu.SemaphoreType.DMA,
            pltpu.SemaphoreType.DMA,
            pltpu.SemaphoreType.DMA,
            pltpu.SemaphoreType.DMA,
            pltpu.SemaphoreType.DMA,
            pltpu.SemaphoreType.DMA,
        ],
    )(_sc_body)
    return fn(dist, feats, coorsT)


def _s3_body(with_coors, g_ref, gc_ref, a_ref, f_ref, c_ref,
             w1b_ref, w1d_ref, w2_ref, b2_ref, wgT_ref, bg_ref,
             wc1_ref, bc1_ref, wc2T_ref, bc2_ref, sc_ref,
             wn1_ref, bn1_ref, wn2_ref, bn2_ref,
             fout_ref, cout_ref, mi_ref):
    ii = lax.broadcasted_iota(jnp.int32, (K, K), 0)
    jj = lax.broadcasted_iota(jnp.int32, (K, K), 1)
    eye = ii == jj
    eye4 = (lax.broadcasted_iota(jnp.int32, (4, 4), 0)
            == lax.broadcasted_iota(jnp.int32, (4, 4), 1))

    rels = []
    rd_rows = []
    rd_cols = []
    for n in range(R3):
        cj = gc_ref[n, :, :]
        ci_row = c_ref[pl.ds(n, 1), :]
        ci_col = jnp.sum(
            jnp.where(eye4, jnp.broadcast_to(ci_row, (4, 4)), 0.0),
            axis=1, keepdims=True)
        rel = ci_col - cj
        rx, ry, rz = rel[0:1, :], rel[1:2, :], rel[2:3, :]
        rd_row = rx * rx + ry * ry + rz * rz
        rd_col = jnp.sum(
            jnp.where(eye, jnp.broadcast_to(rd_row, (K, K)), 0.0),
            axis=1, keepdims=True)
        rels.append((rx, ry, rz))
        rd_rows.append(rd_row)
        rd_cols.append(rd_col)

    a_stack = jnp.concatenate(
        [jnp.broadcast_to(a_ref[pl.ds(n, 1), :], (K, H1)) for n in range(R3)],
        axis=0)
    rd_stack = jnp.concatenate(rd_cols, axis=0)
    h = _lk(jnp.dot(g_ref[...], w1b_ref[...], preferred_element_type=F32)
            + a_stack + rd_stack * w1d_ref[...])
    m = _lk(jnp.dot(h, w2_ref[...], preferred_element_type=F32)
            + b2_ref[...])
    glog = (jnp.sum(m * wgT_ref[...], axis=1, keepdims=True)
            + bg_ref[0])
    m = m * jax.nn.sigmoid(glog)
    if with_coors:
        ch = _lk(jnp.dot(m, wc1_ref[...], preferred_element_type=F32)
                 + bc1_ref[...])
        cw = (jnp.sum(ch * wc2T_ref[...], axis=1, keepdims=True)
              + bc2_ref[0])

    for n in range(R3):
        mi_ref[pl.ds(n, 1), :] = jnp.sum(m[n * K:(n + 1) * K, :], axis=0,
                                         keepdims=True)
        if with_coors:
            rx, ry, rz = rels[n]
            cw_col = cw[n * K:(n + 1) * K, :]
            cw_row = jnp.sum(
                jnp.where(eye, jnp.broadcast_to(cw_col, (K, K)), 0.0),
                axis=0, keepdims=True)
            inv = sc_ref[0] / jnp.clip(
                jnp.sqrt(jnp.clip(rd_rows[n], 1e-12, None)), 1e-8, None)
            w = cw_row * inv
            dx = jnp.sum(w * rx, axis=1, keepdims=True)
            dy = jnp.sum(w * ry, axis=1, keepdims=True)
            dz = jnp.sum(w * rz, axis=1, keepdims=True)
            delta = jnp.concatenate(
                [dx, dy, dz, jnp.zeros((1, 1), F32)], axis=1)
            cout_ref[pl.ds(n, 1), :] = c_ref[pl.ds(n, 1), :] + delta
        else:
            cout_ref[pl.ds(n, 1), :] = c_ref[pl.ds(n, 1), :]

    x = jnp.concatenate([f_ref[...], mi_ref[...]], axis=1)
    nh = _lk(jnp.dot(x, wn1_ref[...], preferred_element_type=F32)
             + bn1_ref[...])
    fout_ref[...] = (jnp.dot(nh, wn2_ref[...], preferred_element_type=F32)
                     + bn2_ref[...] + f_ref[...])


def _s3(with_coors, G, GC, A, feats, coors, w1b, w1d, w2, b2,
        wgT, bg, wc1, bc1, wc2T, bc2, sc, wn1, bn1, wn2, bn2):
    full = lambda shape: pl.BlockSpec(shape, lambda i: tuple(0 for _ in shape))
    smem = pl.BlockSpec(memory_space=pltpu.SMEM)
    return pl.pallas_call(
        functools.partial(_s3_body, with_coors),
        grid=(N // R3,),
        in_specs=[
            pl.BlockSpec((R3 * K, DIM), lambda i: (i, 0)),
            pl.BlockSpec((R3, 4, K), lambda i: (i, 0, 0)),
            pl.BlockSpec((R3, H1), lambda i: (i, 0)),
            pl.BlockSpec((R3, DIM), lambda i: (i, 0)),
            pl.BlockSpec((R3, 4), lambda i: (i, 0)),
            full((DIM, H1)), full((1, H1)),
            full((H1, DIM)), full((1, DIM)), full((1, DIM)), smem,
            full((DIM, 4 * DIM)), full((1, 4 * DIM)), full((1, 4 * DIM)),
            smem, smem,
            full((2 * DIM, 2 * DIM)), full((1, 2 * DIM)),
            full((2 * DIM, DIM)), full((1, DIM)),
        ],
        out_specs=[
            pl.BlockSpec((R3, DIM), lambda i: (i, 0)),
            pl.BlockSpec((R3, 4), lambda i: (i, 0)),
        ],
        out_shape=[
            jax.ShapeDtypeStruct((N, DIM), F32),
            jax.ShapeDtypeStruct((N, 4), F32),
        ],
        scratch_shapes=[pltpu.VMEM((R3, DIM), F32)],
    )(G, GC, A, feats, coors, w1b, w1d, w2, b2, wgT, bg,
      wc1, bc1, wc2T, bc2, sc, wn1, bn1, wn2, bn2)


def kernel(feat_, coor_, batch, We, be, W1, b1, W2, b2, Wg, bg, Wc1, bc1,
           Wc2, bc2, Wn1, bn1, Wn2, bn2, scale):
    feats = _embed(feat_, We, be.reshape(1, DIM))
    coors = jnp.concatenate([coor_, jnp.zeros((N, 1), F32)], axis=1)

    for i in range(LAYERS):
        W1a = W1[i, :DIM]
        W1b = W1[i, DIM:2 * DIM]
        w1d = W1[i, 2 * DIM:2 * DIM + 1]
        coorsT = coors.T
        dist, A = _s1(coors, coorsT, feats, W1a, b1[i].reshape(1, H1))
        G, GC = _s2(dist, feats, coorsT)
        feats, coors = _s3(
            i < LAYERS - 1, G, GC, A, feats, coors,
            W1b, w1d,
            W2[i], b2[i].reshape(1, DIM), Wg[i].T, bg[i],
            Wc1[i], bc1[i].reshape(1, 4 * DIM), Wc2[i].T, bc2[i], scale[i],
            Wn1[i], bn1[i].reshape(1, 2 * DIM),
            Wn2[i], bn2[i].reshape(1, DIM))
    return feats


# speedup vs baseline: 5.1171x; 1.1657x over previous
import functools

import jax
import jax.numpy as jnp
from jax import lax
from jax.experimental import pallas as pl
from jax.experimental.pallas import tpu as pltpu
from jax.experimental.pallas import tpu_sc as plsc

LAYERS = 3
DIM = 128
POS = 3
N = 2048
K = 128
EIN = 2 * DIM + 1
H1 = 2 * EIN
DG = DIM + 16
NW = 32
ROWS_PER_W = N // NW
R1 = 256
R3 = 32
F32 = jnp.float32


def _lk(x):
    return jnp.maximum(x, 0.1 * x)


def _embed_body(f_ref, we_ref, be_ref, o_ref):
    o_ref[...] = _lk(
        jnp.dot(f_ref[...], we_ref[...], preferred_element_type=F32)
        + be_ref[...])


def _embed(feat, We, be2d):
    return pl.pallas_call(
        _embed_body,
        grid=(N // R1,),
        in_specs=[
            pl.BlockSpec((R1, DIM), lambda i: (i, 0)),
            pl.BlockSpec((DIM, DIM), lambda i: (0, 0)),
            pl.BlockSpec((1, DIM), lambda i: (0, 0)),
        ],
        out_specs=pl.BlockSpec((R1, DIM), lambda i: (i, 0)),
        out_shape=jax.ShapeDtypeStruct((N, DIM), F32),
    )(feat, We, be2d)


def _s1_body(c_ref, ct_ref, f_ref, w1a_ref, b1_ref, dist_ref, a_ref):
    c = c_ref[...]
    ct = ct_ref[...]
    d = ((c[:, 0:1] - ct[0:1, :]) ** 2
         + (c[:, 1:2] - ct[1:2, :]) ** 2
         + (c[:, 2:3] - ct[2:3, :]) ** 2)

    bits = lax.bitcast_convert_type(d, jnp.int32)
    lo0 = jnp.zeros((R1, 1), jnp.int32)
    hi0 = jnp.full((R1, 1), 0x7F800000, jnp.int32)
    def step(_, carry):
        lo, hi = carry
        mid = lo + ((hi - lo) >> 1)
        cnt = jnp.sum((bits <= mid).astype(jnp.int32), axis=1, keepdims=True)
        ok = cnt >= K
        return jnp.where(ok, lo, mid + 1), jnp.where(ok, mid, hi)

    _, hi = lax.fori_loop(0, 31, step, (lo0, hi0))
    dist_ref[...] = bits - hi

    a_ref[...] = (jnp.dot(f_ref[...], w1a_ref[...],
                          preferred_element_type=F32) + b1_ref[...])


def _s1(coors, coorsT, feats, W1a, b1_2d):
    return pl.pallas_call(
        _s1_body,
        grid=(N // R1,),
        in_specs=[
            pl.BlockSpec((R1, 4), lambda i: (i, 0)),
            pl.BlockSpec((4, N), lambda i: (0, 0)),
            pl.BlockSpec((R1, DIM), lambda i: (i, 0)),
            pl.BlockSpec((DIM, H1), lambda i: (0, 0)),
            pl.BlockSpec((1, H1), lambda i: (0, 0)),
        ],
        out_specs=[
            pl.BlockSpec((R1, N), lambda i: (i, 0)),
            pl.BlockSpec((R1, H1), lambda i: (i, 0)),
        ],
        out_shape=[
            jax.ShapeDtypeStruct((N, N), jnp.int32),
            jax.ShapeDtypeStruct((N, H1), F32),
        ],
    )(coors, coorsT, feats, W1a, b1_2d)


def _sc_body(dist_hbm, feats_hbm, coorsT_hbm, gf_hbm, gc_hbm,
             dist_v, ltbuf, eqbuf, idxbuf, rows_v, cx_v, cy_v, cz_v,
             cbuf, ld0, ld1, g0, g1, st0, st1, sc0, sc1):
    wid = lax.axis_index("s") * 2 + lax.axis_index("c")
    row0 = wid * ROWS_PER_W
    ld = (ld0, ld1)
    gs = (g0, g1)
    st = (st0, st1)
    sc = (sc0, sc1)
    iot = lax.iota(jnp.int32, 16)
    zero16 = jnp.zeros((16,), jnp.int32)

    pltpu.sync_copy(coorsT_hbm.at[0], cx_v)
    pltpu.sync_copy(coorsT_hbm.at[1], cy_v)
    pltpu.sync_copy(coorsT_hbm.at[2], cz_v)

    pltpu.async_copy(dist_hbm.at[row0], dist_v.at[0], ld[0])
    pltpu.async_copy(dist_hbm.at[row0 + 1], dist_v.at[1], ld[1])

    def finish_row(k, b):
        row = row0 + k
        pltpu.make_async_copy(feats_hbm.at[idxbuf.at[b]], rows_v.at[b],
                              gs[b]).wait()

        @pl.when(k >= 2)
        def _():
            pltpu.make_async_copy(cbuf.at[b], gc_hbm.at[row], sc[b]).wait()

        for kc in range(K // 16):
            idx = idxbuf[b, pl.ds(kc * 16, 16)]
            cbuf[b, 0, pl.ds(kc * 16, 16)] = plsc.load_gather(cx_v, [idx])
            cbuf[b, 1, pl.ds(kc * 16, 16)] = plsc.load_gather(cy_v, [idx])
            cbuf[b, 2, pl.ds(kc * 16, 16)] = plsc.load_gather(cz_v, [idx])

        pltpu.async_copy(rows_v.at[b], gf_hbm.at[pl.ds(row * K, K)], st[b])
        pltpu.async_copy(cbuf.at[b], gc_hbm.at[row], sc[b])

    def row_step(k, b):
        row = row0 + k
        pltpu.make_async_copy(dist_hbm.at[row], dist_v.at[b], ld[b]).wait()

        @plsc.parallel_loop(0, N // 16, 1, unroll=4, carry=(zero16, zero16))
        def chunk(cc, carry):
            nlt_v, neq_v = carry
            d = dist_v[b, pl.ds(cc * 16, 16)]
            col = iot + cc * 16
            is_lt = d < 0
            is_eq = d == 0
            clt = plsc.cumsum(jnp.where(is_lt, 1, 0))
            ceq = plsc.cumsum(jnp.where(is_eq, 1, 0))
            plsc.store_scatter(ltbuf, [nlt_v + clt - 1], col, mask=is_lt)
            plsc.store_scatter(eqbuf, [neq_v + ceq - 1], col, mask=is_eq)
            nlt_v = nlt_v + plsc.all_reduce_population_count(is_lt)
            neq_v = neq_v + plsc.all_reduce_population_count(is_eq)
            return nlt_v, neq_v

        nlt_v, _ = chunk

        for kc in range(K // 16):
            v = eqbuf[pl.ds(kc * 16, 16)]
            plsc.store_scatter(ltbuf, [nlt_v + kc * 16 + iot], v)
        for kc in range(K // 16):
            idxbuf[b, pl.ds(kc * 16, 16)] = ltbuf[pl.ds(kc * 16, 16)]

        @pl.when(k >= 2)
        def _():
            pltpu.make_async_copy(rows_v.at[b], gf_hbm.at[pl.ds(0, K)],
                                  st[b]).wait()

        pltpu.async_copy(feats_hbm.at[idxbuf.at[b]], rows_v.at[b], gs[b])

        @pl.when(k + 2 < ROWS_PER_W)
        def _():
            pltpu.async_copy(dist_hbm.at[row + 2], dist_v.at[b], ld[b])

        @pl.when(k >= 1)
        def _():
            finish_row(k - 1, b ^ 1)

    def pair(i, _):
        row_step(2 * i, 0)
        row_step(2 * i + 1, 1)
        return 0

    lax.fori_loop(0, ROWS_PER_W // 2, pair, 0)
    finish_row(ROWS_PER_W - 1, 1)
    pltpu.make_async_copy(rows_v.at[0], gf_hbm.at[pl.ds(0, K)], st[0]).wait()
    pltpu.make_async_copy(rows_v.at[1], gf_hbm.at[pl.ds(0, K)], st[1]).wait()
    pltpu.make_async_copy(cbuf.at[0], gc_hbm.at[0], sc[0]).wait()
    pltpu.make_async_copy(cbuf.at[1], gc_hbm.at[0], sc[1]).wait()


def _s2(dist, feats, coorsT):
    mesh = plsc.VectorSubcoreMesh(core_axis_name="c", subcore_axis_name="s")
    fn = functools.partial(
        pl.kernel, mesh=mesh,
        compiler_params=pltpu.CompilerParams(needs_layout_passes=False),
        out_type=[
            jax.ShapeDtypeStruct((N * K, DIM), F32),
            jax.ShapeDtypeStruct((N, 4, K), F32),
        ],
        scratch_types=[
            pltpu.VMEM((2, N), jnp.int32),
            pltpu.VMEM((272,), jnp.int32),
            pltpu.VMEM((N + 16,), jnp.int32),
            pltpu.VMEM((2, K), jnp.int32),
            pltpu.VMEM((2, K, DIM), F32),
            pltpu.VMEM((N,), F32),
            pltpu.VMEM((N,), F32),
            pltpu.VMEM((N,), F32),
            pltpu.VMEM((2, 4, K), F32),
            pltpu.SemaphoreType.DMA,
            pltpu.SemaphoreType.DMA,
            pltpu.SemaphoreType.DMA,
            pltpu.SemaphoreType.DMA,
            pltpu.SemaphoreType.DMA,
            pltpu.SemaphoreType.DMA,
            pltpu.SemaphoreType.DMA,
            pltpu.SemaphoreType.DMA,
        ],
    )(_sc_body)
    return fn(dist, feats, coorsT)


def _s3_body(with_coors, g_ref, gc_ref, a_ref, f_ref, c_ref,
             w1b_ref, w1d_ref, w2_ref, b2_ref, wgT_ref, bg_ref,
             wc1_ref, bc1_ref, wc2T_ref, bc2_ref, sc_ref,
             wn1_ref, bn1_ref, wn2_ref, bn2_ref,
             fout_ref, cout_ref, mi_ref):
    ii = lax.broadcasted_iota(jnp.int32, (K, K), 0)
    jj = lax.broadcasted_iota(jnp.int32, (K, K), 1)
    eye = ii == jj
    eye4 = (lax.broadcasted_iota(jnp.int32, (4, 4), 0)
            == lax.broadcasted_iota(jnp.int32, (4, 4), 1))

    rels = []
    rd_rows = []
    rd_cols = []
    for n in range(R3):
        cj = gc_ref[n, :, :]
        ci_row = c_ref[pl.ds(n, 1), :]
        ci_col = jnp.sum(
            jnp.where(eye4, jnp.broadcast_to(ci_row, (4, 4)), 0.0),
            axis=1, keepdims=True)
        rel = ci_col - cj
        rx, ry, rz = rel[0:1, :], rel[1:2, :], rel[2:3, :]
        rd_row = rx * rx + ry * ry + rz * rz
        rd_col = jnp.sum(
            jnp.where(eye, jnp.broadcast_to(rd_row, (K, K)), 0.0),
            axis=1, keepdims=True)
        rels.append((rx, ry, rz))
        rd_rows.append(rd_row)
        rd_cols.append(rd_col)

    a_stack = jnp.concatenate(
        [jnp.broadcast_to(a_ref[pl.ds(n, 1), :], (K, H1)) for n in range(R3)],
        axis=0)
    rd_stack = jnp.concatenate(rd_cols, axis=0)
    h = _lk(jnp.dot(g_ref[...], w1b_ref[...], preferred_element_type=F32)
            + a_stack + rd_stack * w1d_ref[...])
    m = _lk(jnp.dot(h, w2_ref[...], preferred_element_type=F32)
            + b2_ref[...])
    glog = (jnp.sum(m * wgT_ref[...], axis=1, keepdims=True)
            + bg_ref[0])
    m = m * jax.nn.sigmoid(glog)
    if with_coors:
        ch = _lk(jnp.dot(m, wc1_ref[...], preferred_element_type=F32)
                 + bc1_ref[...])
        cw = (jnp.sum(ch * wc2T_ref[...], axis=1, keepdims=True)
              + bc2_ref[0])

    for n in range(R3):
        mi_ref[pl.ds(n, 1), :] = jnp.sum(m[n * K:(n + 1) * K, :], axis=0,
                                         keepdims=True)
        if with_coors:
            rx, ry, rz = rels[n]
            cw_col = cw[n * K:(n + 1) * K, :]
            cw_row = jnp.sum(
                jnp.where(eye, jnp.broadcast_to(cw_col, (K, K)), 0.0),
                axis=0, keepdims=True)
            inv = sc_ref[0] / jnp.clip(
                jnp.sqrt(jnp.clip(rd_rows[n], 1e-12, None)), 1e-8, None)
            w = cw_row * inv
            dx = jnp.sum(w * rx, axis=1, keepdims=True)
            dy = jnp.sum(w * ry, axis=1, keepdims=True)
            dz = jnp.sum(w * rz, axis=1, keepdims=True)
            delta = jnp.concatenate(
                [dx, dy, dz, jnp.zeros((1, 1), F32)], axis=1)
            cout_ref[pl.ds(n, 1), :] = c_ref[pl.ds(n, 1), :] + delta
        else:
            cout_ref[pl.ds(n, 1), :] = c_ref[pl.ds(n, 1), :]

    x = jnp.concatenate([f_ref[...], mi_ref[...]], axis=1)
    nh = _lk(jnp.dot(x, wn1_ref[...], preferred_element_type=F32)
             + bn1_ref[...])
    fout_ref[...] = (jnp.dot(nh, wn2_ref[...], preferred_element_type=F32)
                     + bn2_ref[...] + f_ref[...])


def _s3(with_coors, G, GC, A, feats, coors, w1b, w1d, w2, b2,
        wgT, bg, wc1, bc1, wc2T, bc2, sc, wn1, bn1, wn2, bn2):
    full = lambda shape: pl.BlockSpec(shape, lambda i: tuple(0 for _ in shape))
    smem = pl.BlockSpec(memory_space=pltpu.SMEM)
    return pl.pallas_call(
        functools.partial(_s3_body, with_coors),
        grid=(N // R3,),
        in_specs=[
            pl.BlockSpec((R3 * K, DIM), lambda i: (i, 0)),
            pl.BlockSpec((R3, 4, K), lambda i: (i, 0, 0)),
            pl.BlockSpec((R3, H1), lambda i: (i, 0)),
            pl.BlockSpec((R3, DIM), lambda i: (i, 0)),
            pl.BlockSpec((R3, 4), lambda i: (i, 0)),
            full((DIM, H1)), full((1, H1)),
            full((H1, DIM)), full((1, DIM)), full((1, DIM)), smem,
            full((DIM, 4 * DIM)), full((1, 4 * DIM)), full((1, 4 * DIM)),
            smem, smem,
            full((2 * DIM, 2 * DIM)), full((1, 2 * DIM)),
            full((2 * DIM, DIM)), full((1, DIM)),
        ],
        out_specs=[
            pl.BlockSpec((R3, DIM), lambda i: (i, 0)),
            pl.BlockSpec((R3, 4), lambda i: (i, 0)),
        ],
        out_shape=[
            jax.ShapeDtypeStruct((N, DIM), F32),
            jax.ShapeDtypeStruct((N, 4), F32),
        ],
        scratch_shapes=[pltpu.VMEM((R3, DIM), F32)],
    )(G, GC, A, feats, coors, w1b, w1d, w2, b2, wgT, bg,
      wc1, bc1, wc2T, bc2, sc, wn1, bn1, wn2, bn2)


def kernel(feat_, coor_, batch, We, be, W1, b1, W2, b2, Wg, bg, Wc1, bc1,
           Wc2, bc2, Wn1, bn1, Wn2, bn2, scale):
    feats = _embed(feat_, We, be.reshape(1, DIM))
    coors = jnp.concatenate([coor_, jnp.zeros((N, 1), F32)], axis=1)

    for i in range(LAYERS):
        W1a = W1[i, :DIM]
        W1b = W1[i, DIM:2 * DIM]
        w1d = W1[i, 2 * DIM:2 * DIM + 1]
        coorsT = coors.T
        dist, A = _s1(coors, coorsT, feats, W1a, b1[i].reshape(1, H1))
        G, GC = _s2(dist, feats, coorsT)
        feats, coors = _s3(
            i < LAYERS - 1, G, GC, A, feats, coors,
            W1b, w1d,
            W2[i], b2[i].reshape(1, DIM), Wg[i].T, bg[i],
            Wc1[i], bc1[i].reshape(1, 4 * DIM), Wc2[i].T, bc2[i], scale[i],
            Wn1[i], bn1[i].reshape(1, 2 * DIM),
            Wn2[i], bn2[i].reshape(1, DIM))
    return feats
